# Initial kernel scaffold; baseline (speedup 1.0000x reference)
import functools

import jax
import jax.numpy as jnp
from jax import lax
from jax.experimental import pallas as pl
from jax.experimental.pallas import tpu as pltpu
from jax.experimental.pallas import tpu_sc as plsc

N = 320000
D = 128
S = 10000
WS = 16
NWIN = S // WS
NW = 32
C = 64
LANES = 16
BIG = jnp.float32(3.0e38)

_mesh = plsc.VectorSubcoreMesh(core_axis_name="c", subcore_axis_name="s")


@functools.partial(
    pl.kernel,
    out_type=jax.ShapeDtypeStruct((S, 3 * D), jnp.float32),
    mesh=_mesh,
    scratch_types=[
        pltpu.VMEM((NWIN + 15,), jnp.int32),
        pltpu.VMEM((C, D), jnp.float32),
        pltpu.VMEM((C,), jnp.int32),
        pltpu.VMEM((WS, 3 * D), jnp.float32),
        pltpu.VMEM((WS,), jnp.float32),
    ],
)
def _sc_aggr(x_hbm, b_hbm, ws_hbm, out_hbm, ws_v, xbuf, bbuf, acc, cnt):
    cid = lax.axis_index("c")
    sid = lax.axis_index("s")
    wid = sid * 2 + cid

    pltpu.sync_copy(ws_hbm, ws_v)

    w0 = (wid * NWIN) // NW
    w1 = ((wid + 1) * NWIN) // NW

    def win_body(w, _):
        wbase = w * WS
        rs = ws_v[w]
        re = ws_v[w + 1]

        zeros = jnp.zeros((LANES,), jnp.float32)
        for l in range(WS):
            for c in range(D // LANES):
                acc[l, pl.ds(c * LANES, LANES)] = zeros
                acc[l, pl.ds(D + c * LANES, LANES)] = zeros - BIG
                acc[l, pl.ds(2 * D + c * LANES, LANES)] = zeros + BIG
        cnt[...] = jnp.zeros((WS,), jnp.float32)

        rs8 = rs & jnp.int32(-8)
        nch = (re - rs8 + (C - 1)) // C

        def chunk_body(k, _):
            rb = rs8 + k * C
            b = jnp.minimum(rb, N - C)
            pltpu.sync_copy(x_hbm.at[pl.ds(b, C), :], xbuf)
            pltpu.sync_copy(b_hbm.at[pl.ds(b, C)], bbuf)
            lo = jnp.maximum(rs, rb) - b
            hi = jnp.minimum(re, rb + C) - b

            def row_body(j, _):
                l = bbuf[j] - wbase
                for c in range(D // LANES):
                    sl = pl.ds(c * LANES, LANES)
                    v = xbuf[j, sl]
                    plsc.addupdate(acc.at[l, sl], v)
                    msl = pl.ds(D + c * LANES, LANES)
                    acc[l, msl] = jnp.maximum(acc[l, msl], v)
                    nsl = pl.ds(2 * D + c * LANES, LANES)
                    acc[l, nsl] = jnp.minimum(acc[l, nsl], v)
                cnt[l] = cnt[l] + 1.0
                return 0

            lax.fori_loop(lo, hi, row_body, 0)
            return 0

        lax.fori_loop(0, nch, chunk_body, 0)

        for l in range(WS):
            c_l = cnt[l]
            rc = 1.0 / jnp.maximum(c_l, 1.0)
            nz = jnp.broadcast_to(c_l > 0.0, (LANES,))
            for c in range(D // LANES):
                sl = pl.ds(c * LANES, LANES)
                acc[l, sl] = acc[l, sl] * rc
                msl = pl.ds(D + c * LANES, LANES)
                acc[l, msl] = jnp.where(nz, acc[l, msl], 0.0)
                nsl = pl.ds(2 * D + c * LANES, LANES)
                acc[l, nsl] = jnp.where(nz, acc[l, nsl], 0.0)

        pltpu.sync_copy(acc, out_hbm.at[pl.ds(wbase, WS), :])
        return 0

    lax.fori_loop(w0, w1, win_body, 0)


def kernel(x, batch):
    b32 = batch.astype(jnp.int32)
    bounds = jnp.arange(NWIN + 1, dtype=jnp.int32) * WS
    ws = jnp.searchsorted(b32, bounds).astype(jnp.int32)
    ws_pad = jnp.concatenate([ws, jnp.full((14,), N, jnp.int32)])
    return _sc_aggr(x, b32, ws_pad)


# speedup vs baseline: 2.7200x; 2.7200x over previous
import functools

import jax
import jax.numpy as jnp
from jax import lax
from jax.experimental import pallas as pl
from jax.experimental.pallas import tpu as pltpu
from jax.experimental.pallas import tpu_sc as plsc

N = 320000
D = 128
S = 10000
WS = 16
NWIN = S // WS
NW = 32
C = 64
LANES = 16
BIG = 3.0e38

_mesh = plsc.VectorSubcoreMesh(core_axis_name="c", subcore_axis_name="s")


@functools.partial(
    pl.kernel,
    out_type=jax.ShapeDtypeStruct((S, 3 * D), jnp.float32),
    mesh=_mesh,
    scratch_types=[
        pltpu.VMEM((NWIN + 31,), jnp.int32),
        pltpu.VMEM((C, D), jnp.float32),
        pltpu.VMEM((C + LANES,), jnp.int32),
        pltpu.VMEM((WS, 3 * D), jnp.float32),
        pltpu.VMEM((WS,), jnp.float32),
    ],
)
def _sc_aggr(x_hbm, b_hbm, ws_hbm, out_hbm, ws_v, xbuf, bbuf, acc, cnt):
    cid = lax.axis_index("c")
    sid = lax.axis_index("s")
    wid = sid * 2 + cid

    pltpu.sync_copy(ws_hbm, ws_v)

    w0 = (wid * NWIN) // NW
    w1 = ((wid + 1) * NWIN) // NW

    def win_body(w, _):
        wbase = w * WS
        wsv = ws_v[pl.ds(w, LANES)]
        rs = wsv[0]
        re = wsv[1]

        zeros = jnp.zeros((LANES,), jnp.float32)
        for l in range(WS):
            for c in range(D // LANES):
                acc[l, pl.ds(c * LANES, LANES)] = zeros
                acc[l, pl.ds(D + c * LANES, LANES)] = zeros - BIG
                acc[l, pl.ds(2 * D + c * LANES, LANES)] = zeros + BIG
        cnt[...] = jnp.zeros((WS,), jnp.float32)

        rs8 = rs & jnp.int32(-8)
        nch = (re - rs8 + (C - 1)) // C

        def chunk_body(k, _):
            rb = rs8 + k * C
            b = pl.multiple_of(jnp.minimum(rb, N - C), 8)
            pltpu.sync_copy(x_hbm.at[pl.ds(b, C), :], xbuf)
            pltpu.sync_copy(b_hbm.at[pl.ds(b, C)], bbuf.at[pl.ds(0, C)])
            lo = jnp.maximum(rs, rb) - b
            hi = jnp.minimum(re, rb + C) - b
            lane = lax.iota(jnp.int32, LANES)

            def row_body(j, _):
                l = bbuf[pl.ds(j, LANES)][0] - wbase
                for c in range(D // LANES):
                    sl = pl.ds(c * LANES, LANES)
                    v = xbuf[j, sl]
                    plsc.addupdate(acc.at[l, sl], v)
                    msl = pl.ds(D + c * LANES, LANES)
                    acc[l, msl] = jnp.maximum(acc[l, msl], v)
                    nsl = pl.ds(2 * D + c * LANES, LANES)
                    acc[l, nsl] = jnp.minimum(acc[l, nsl], v)
                cnt[...] = cnt[...] + jnp.where(lane == l, 1.0, 0.0)
                return 0

            lax.fori_loop(lo, hi, row_body, 0)
            return 0

        lax.fori_loop(0, nch, chunk_body, 0)

        cv = cnt[...]
        rcv = 1.0 / jnp.maximum(cv, 1.0)
        nzv = jnp.where(cv > 0.0, 1.0, 0.0)
        for l in range(WS):
            rc = rcv[l]
            nz = nzv[l]
            for c in range(D // LANES):
                sl = pl.ds(c * LANES, LANES)
                acc[l, sl] = acc[l, sl] * rc
                msl = pl.ds(D + c * LANES, LANES)
                acc[l, msl] = acc[l, msl] * nz
                nsl = pl.ds(2 * D + c * LANES, LANES)
                acc[l, nsl] = acc[l, nsl] * nz

        pltpu.sync_copy(acc, out_hbm.at[pl.ds(wbase, WS), :])
        return 0

    lax.fori_loop(w0, w1, win_body, 0)


def kernel(x, batch):
    b32 = batch.astype(jnp.int32)
    bounds = jnp.arange(NWIN + 1, dtype=jnp.int32) * WS
    ws = jnp.searchsorted(b32, bounds).astype(jnp.int32)
    ws_pad = jnp.concatenate([ws, jnp.full((30,), N, jnp.int32)])
    return _sc_aggr(x, b32, ws_pad)


# speedup vs baseline: 3.8561x; 1.4177x over previous
import dataclasses
import functools

import jax
import jax.numpy as jnp
from jax import lax
from jax.experimental import pallas as pl
from jax.experimental.pallas import tpu as pltpu
from jax.experimental.pallas import tpu_sc as plsc

N = 320000
D = 128
S = 10000
WS = 16
NWIN = S // WS
NW = 32
C = 128
LANES = 16
NSL = D // LANES
BIG = 3.0e38

_mesh = plsc.VectorSubcoreMesh(core_axis_name="c", subcore_axis_name="s")

_cp = pltpu.CompilerParams()
if "needs_layout_passes" in pltpu.CompilerParams.__dataclass_fields__:
    _cp = dataclasses.replace(_cp, needs_layout_passes=False)


@functools.partial(
    pl.kernel,
    out_type=jax.ShapeDtypeStruct((S, 3 * D), jnp.float32),
    mesh=_mesh,
    compiler_params=_cp,
    scratch_types=[
        pltpu.VMEM((NWIN + 31,), jnp.int32),
        pltpu.VMEM((2, C, D), jnp.float32),
        pltpu.VMEM((2, C), jnp.int32),
        pltpu.VMEM((WS + 1, 3 * D), jnp.float32),
        pltpu.VMEM((WS + 1,), jnp.float32),
        pltpu.SemaphoreType.DMA,
        pltpu.SemaphoreType.DMA,
    ],
)
def _sc_aggr(x_hbm, b_hbm, ws_hbm, out_hbm, ws_v, xbuf, bbuf, acc, cnt,
             sem0, sem1):
    cid = lax.axis_index("c")
    sid = lax.axis_index("s")
    wid = sid * 2 + cid
    sems = (sem0, sem1)

    pltpu.sync_copy(ws_hbm, ws_v)

    w0 = (wid * NWIN) // NW
    w1 = ((wid + 1) * NWIN) // NW

    def start_dma(m, b):
        off = pl.multiple_of(m * C, 8)
        pltpu.async_copy(x_hbm.at[pl.ds(off, C), :], xbuf.at[b], sems[b])
        pltpu.async_copy(b_hbm.at[pl.ds(off, C)], bbuf.at[b], sems[b])

    def wait_dma(b):
        pltpu.make_async_copy(x_hbm.at[pl.ds(0, C), :], xbuf.at[b],
                              sems[b]).wait()
        pltpu.make_async_copy(b_hbm.at[pl.ds(0, C)], bbuf.at[b],
                              sems[b]).wait()

    def win_body(w, _):
        wbase = w * WS
        wsv = ws_v[pl.ds(w, LANES)]
        rs = wsv[0]
        re = wsv[1]

        zeros = jnp.zeros((LANES,), jnp.float32)
        ones = jnp.ones((LANES,), jnp.float32)
        for l in range(WS):
            for c in range(NSL):
                acc[l, pl.ds(c * LANES, LANES)] = zeros
                acc[l, pl.ds(D + c * LANES, LANES)] = zeros - BIG
                acc[l, pl.ds(2 * D + c * LANES, LANES)] = zeros + BIG
        cnt[pl.ds(0, LANES)] = zeros

        m0 = rs // C
        m1 = (re + (C - 1)) // C
        nch = m1 - m0

        for b in range(2):
            @pl.when(m0 + b < m1)
            def _():
                start_dma(m0 + b, b)

        def process(m, b):
            base = m * C
            lo = jnp.maximum(rs - base, 0)
            hi = jnp.minimum(re - base, C)
            g0 = lo // LANES
            g1 = (hi + (LANES - 1)) // LANES

            def group(g, _):
                p = g * LANES
                bvec = bbuf.at[b][pl.ds(p, LANES)]
                lvec = bvec - wbase
                okv = (lvec >= 0) & (lvec < WS)
                lcl = jnp.where(okv, lvec, WS)
                plsc.addupdate_scatter(cnt, [lcl], ones)
                for j in range(LANES):
                    l = lcl[j]
                    for c in range(NSL):
                        sl = pl.ds(c * LANES, LANES)
                        v = xbuf.at[b][p + j, sl]
                        plsc.addupdate(acc.at[l, sl], v)
                        msl = pl.ds(D + c * LANES, LANES)
                        acc[l, msl] = jnp.maximum(acc[l, msl], v)
                        nsl2 = pl.ds(2 * D + c * LANES, LANES)
                        acc[l, nsl2] = jnp.minimum(acc[l, nsl2], v)
                return 0

            lax.fori_loop(g0, g1, group, 0)

        def outer(i, _):
            k2 = m0 + 2 * i
            for b in range(2):
                m = k2 + b

                @pl.when(m < m1)
                def _():
                    wait_dma(b)
                    process(m, b)

                    @pl.when(m + 2 < m1)
                    def _():
                        start_dma(m + 2, b)
            return 0

        lax.fori_loop(0, (nch + 1) // 2, outer, 0)

        cv = cnt[pl.ds(0, LANES)]
        rcv = 1.0 / jnp.maximum(cv, 1.0)
        nzv = jnp.where(cv > 0.0, 1.0, 0.0)
        for l in range(WS):
            rc = rcv[l]
            nz = nzv[l]
            for c in range(NSL):
                sl = pl.ds(c * LANES, LANES)
                acc[l, sl] = acc[l, sl] * rc
                msl = pl.ds(D + c * LANES, LANES)
                acc[l, msl] = acc[l, msl] * nz
                nsl2 = pl.ds(2 * D + c * LANES, LANES)
                acc[l, nsl2] = acc[l, nsl2] * nz

        pltpu.sync_copy(acc.at[pl.ds(0, WS), :], out_hbm.at[pl.ds(wbase, WS), :])
        return 0

    lax.fori_loop(w0, w1, win_body, 0)


def kernel(x, batch):
    b32 = batch.astype(jnp.int32)
    bounds = jnp.arange(NWIN + 1, dtype=jnp.int32) * WS
    ws = jnp.searchsorted(b32, bounds).astype(jnp.int32)
    ws_pad = jnp.concatenate([ws, jnp.full((30,), N, jnp.int32)])
    return _sc_aggr(x, b32, ws_pad)


# speedup vs baseline: 4.6254x; 1.1995x over previous
import dataclasses
import functools

import jax
import jax.numpy as jnp
from jax import lax
from jax.experimental import pallas as pl
from jax.experimental.pallas import tpu as pltpu
from jax.experimental.pallas import tpu_sc as plsc

N = 320000
D = 128
S = 10000
WS = 16
NWIN = S // WS
NW = 32
C = 128
LANES = 16
NSL = D // LANES
BIG = 3.0e38

_mesh = plsc.VectorSubcoreMesh(core_axis_name="c", subcore_axis_name="s")

_cp = pltpu.CompilerParams()
if "needs_layout_passes" in pltpu.CompilerParams.__dataclass_fields__:
    _cp = dataclasses.replace(_cp, needs_layout_passes=False)


@functools.partial(
    pl.kernel,
    out_type=jax.ShapeDtypeStruct((S, 3 * D), jnp.float32),
    mesh=_mesh,
    compiler_params=_cp,
    scratch_types=[
        pltpu.VMEM((NWIN + 31,), jnp.int32),
        pltpu.VMEM((C, D), jnp.float32),
        pltpu.VMEM((C, D), jnp.float32),
        pltpu.VMEM((2 * C,), jnp.int32),
        pltpu.VMEM((2 * C,), jnp.int32),
        pltpu.VMEM((WS, 3 * D), jnp.float32),
        pltpu.SemaphoreType.DMA,
        pltpu.SemaphoreType.DMA,
    ],
)
def _sc_aggr(x_hbm, b_hbm, ws_hbm, out_hbm, ws_v, xbuf0, xbuf1, bbuf0,
             bbuf1, acc, sem0, sem1):
    cid = lax.axis_index("c")
    sid = lax.axis_index("s")
    wid = sid * 2 + cid
    sems = (sem0, sem1)
    xbufs = (xbuf0, xbuf1)
    bbufs = (bbuf0, bbuf1)

    pltpu.sync_copy(ws_hbm, ws_v)

    w0 = (wid * NWIN) // NW
    w1 = ((wid + 1) * NWIN) // NW

    zerov = jnp.zeros((LANES,), jnp.float32)
    negbig = zerov - BIG
    posbig = zerov + BIG

    def start_dma(m, b):
        off = pl.multiple_of(jnp.minimum(m * C, N - C), 8)
        pltpu.async_copy(x_hbm.at[pl.ds(off, C), :], xbufs[b], sems[b])
        pltpu.async_copy(b_hbm.at[pl.ds(off, C)],
                         bbufs[b].at[pl.ds(0, C)], sems[b])

    def wait_dma(b):
        pltpu.make_async_copy(x_hbm.at[pl.ds(0, C), :], xbufs[b],
                              sems[b]).wait()
        pltpu.make_async_copy(b_hbm.at[pl.ds(0, C)],
                              bbufs[b].at[pl.ds(0, C)], sems[b]).wait()

    def win_body(w, _):
        wbase = w * WS
        wsv = ws_v[pl.ds(w, LANES)]
        rs = wsv[0]
        re = wsv[1]

        for l in range(WS):
            for c in range(3 * NSL):
                acc[l, pl.ds(c * LANES, LANES)] = zerov

        m0 = rs // C
        m1 = (re + (C - 1)) // C
        nch = m1 - m0
        npair = (nch + 1) // 2

        @pl.when(m0 < m1)
        def _():
            start_dma(m0, 0)

        @pl.when(m0 + 1 < m1)
        def _():
            start_dma(m0 + 1, 1)

        def flush(lc, cr, regs):
            cntv = jnp.broadcast_to(cr, (LANES,))
            for c in range(NSL):
                acc[lc, pl.ds(c * LANES, LANES)] = regs[c] / cntv
                acc[lc, pl.ds(D + c * LANES, LANES)] = regs[NSL + c]
                acc[lc, pl.ds(2 * D + c * LANES, LANES)] = regs[2 * NSL + c]

        def make_row_body(b):
            xb = xbufs[b]
            bb = bbufs[b]

            def row_body(q, carry):
                lc, cr = carry[0], carry[1]
                regs = carry[2:]
                l_row = bb[pl.ds(q, LANES)][0] - wbase
                changed = l_row != lc

                @pl.when(changed & (lc >= 0))
                def _():
                    flush(lc, cr, regs)

                chv = jnp.broadcast_to(changed, (LANES,))
                kv = jnp.where(chv, 0.0, 1.0)
                new = [l_row, jnp.where(changed, 1.0, cr + 1.0)]
                for c in range(NSL):
                    v = xb[q, pl.ds(c * LANES, LANES)]
                    new.append(regs[c] * kv + v)
                for c in range(NSL):
                    v = xb[q, pl.ds(c * LANES, LANES)]
                    new.append(jnp.maximum(
                        jnp.where(chv, negbig, regs[NSL + c]), v))
                for c in range(NSL):
                    v = xb[q, pl.ds(c * LANES, LANES)]
                    new.append(jnp.minimum(
                        jnp.where(chv, posbig, regs[2 * NSL + c]), v))
                return tuple(new)
            return row_body

        row_bodies = (make_row_body(0), make_row_body(1))

        def process(m, b, carry):
            @pl.when(m < m1)
            def _():
                wait_dma(b)

            base = m * C
            lo = jnp.maximum(rs - base, 0)
            hi = jnp.minimum(re - base, C)
            carry = lax.fori_loop(lo, hi, row_bodies[b], carry)

            @pl.when(m + 2 < m1)
            def _():
                start_dma(m + 2, b)

            return carry

        init = (jnp.int32(-1), jnp.float32(0.0)) + (zerov,) * (3 * NSL)

        def pair_body(i, carry):
            m = m0 + 2 * i
            carry = process(m, 0, carry)
            carry = process(m + 1, 1, carry)
            return carry

        carry = lax.fori_loop(0, npair, pair_body, init)

        lc, cr = carry[0], carry[1]

        @pl.when(lc >= 0)
        def _():
            flush(lc, cr, carry[2:])

        pltpu.sync_copy(acc, out_hbm.at[pl.ds(wbase, WS), :])
        return 0

    lax.fori_loop(w0, w1, win_body, 0)


def kernel(x, batch):
    b32 = batch.astype(jnp.int32)
    bounds = jnp.arange(NWIN + 1, dtype=jnp.int32) * WS
    ws = jnp.searchsorted(b32, bounds).astype(jnp.int32)
    ws_pad = jnp.concatenate([ws, jnp.full((30,), N, jnp.int32)])
    return _sc_aggr(x, b32, ws_pad)
